# baseline (speedup 1.0000x reference)
import functools

import jax
import jax.numpy as jnp
from jax import lax
from jax.experimental import pallas as pl
from jax.experimental.pallas import tpu as pltpu
from jax.experimental.pallas import tpu_sc as plsc

_B = 16384
_MF_D = 32
_MLP_D = 64
_NC, _NS = 2, 16
_NW = _NC * _NS
_BPW = _B // _NW
_CHUNK = 128
_NCHUNK = _BPW // _CHUNK

_TC_BLK = 2048


def _sc_gather_kernel(uidx_hbm, iidx_hbm, mfu_hbm, mfi_hbm, mlu_hbm, mli_hbm,
                      out_mfu, out_mfi, out_mlu, out_mli,
                      uidx_v, iidx_v, mfu_v, mfi_v, mlu_v, mli_v, sem):
    wid = lax.axis_index("s") * _NC + lax.axis_index("c")
    base = wid * _BPW
    pltpu.sync_copy(uidx_hbm.at[wid], uidx_v)
    pltpu.sync_copy(iidx_hbm.at[wid], iidx_v)
    copies = []
    for j in range(_NCHUNK):
        off = j * _CHUNK
        copies.append(pltpu.async_copy(
            mfu_hbm.at[uidx_v.at[j]], mfu_v.at[pl.ds(off, _CHUNK)], sem))
        copies.append(pltpu.async_copy(
            mfi_hbm.at[iidx_v.at[j]], mfi_v.at[pl.ds(off, _CHUNK)], sem))
        copies.append(pltpu.async_copy(
            mlu_hbm.at[uidx_v.at[j]], mlu_v.at[pl.ds(off, _CHUNK)], sem))
        copies.append(pltpu.async_copy(
            mli_hbm.at[iidx_v.at[j]], mli_v.at[pl.ds(off, _CHUNK)], sem))
    for cp in copies:
        cp.wait()
    pltpu.sync_copy(mfu_v, out_mfu.at[pl.ds(base, _BPW)])
    pltpu.sync_copy(mfi_v, out_mfi.at[pl.ds(base, _BPW)])
    pltpu.sync_copy(mlu_v, out_mlu.at[pl.ds(base, _BPW)])
    pltpu.sync_copy(mli_v, out_mli.at[pl.ds(base, _BPW)])


_sc_gather = functools.partial(
    pl.kernel,
    mesh=plsc.VectorSubcoreMesh(core_axis_name="c", subcore_axis_name="s"),
    out_type=[
        jax.ShapeDtypeStruct((_B, _MF_D), jnp.float32),
        jax.ShapeDtypeStruct((_B, _MF_D), jnp.float32),
        jax.ShapeDtypeStruct((_B, _MLP_D), jnp.float32),
        jax.ShapeDtypeStruct((_B, _MLP_D), jnp.float32),
    ],
    scratch_types=[
        pltpu.VMEM((_NCHUNK, _CHUNK), jnp.int32),
        pltpu.VMEM((_NCHUNK, _CHUNK), jnp.int32),
        pltpu.VMEM((_BPW, _MF_D), jnp.float32),
        pltpu.VMEM((_BPW, _MF_D), jnp.float32),
        pltpu.VMEM((_BPW, _MLP_D), jnp.float32),
        pltpu.VMEM((_BPW, _MLP_D), jnp.float32),
        pltpu.SemaphoreType.DMA,
    ],
    compiler_params=pltpu.CompilerParams(use_tc_tiling_on_sc=False),
)(_sc_gather_kernel)


def _tc_mlp_kernel(mfu, mfi, mlu, mli, w1u, w1i, b1, w2, b2, wfm, wfh, bf, out):
    h = jnp.dot(mlu[...], w1u[...], preferred_element_type=jnp.float32)
    h = h + jnp.dot(mli[...], w1i[...], preferred_element_type=jnp.float32)
    h = jnp.maximum(h + b1[...], 0.0)
    h = jnp.dot(h, w2[...], preferred_element_type=jnp.float32) + b2[...]
    h = jnp.maximum(h, 0.0)
    mf = mfu[...] * mfi[...]
    out[...] = (jnp.dot(mf, wfm[...], preferred_element_type=jnp.float32)
                + jnp.dot(h, wfh[...], preferred_element_type=jnp.float32)
                + bf[...])


def _tc_mlp(mfu, mfi, mlu, mli, w1u, w1i, b1, w2, b2, wfm, wfh, bf):
    grid = _B // _TC_BLK
    row_spec = lambda d: pl.BlockSpec((_TC_BLK, d), lambda i: (i, 0))
    full = lambda a: pl.BlockSpec(a.shape, lambda i: (0,) * a.ndim)
    return pl.pallas_call(
        _tc_mlp_kernel,
        grid=(grid,),
        in_specs=[
            row_spec(_MF_D), row_spec(_MF_D), row_spec(_MLP_D), row_spec(_MLP_D),
            full(w1u), full(w1i), full(b1), full(w2), full(b2),
            full(wfm), full(wfh), full(bf),
        ],
        out_specs=pl.BlockSpec((_TC_BLK, 1), lambda i: (i, 0)),
        out_shape=jax.ShapeDtypeStruct((_B, 1), jnp.float32),
    )(mfu, mfi, mlu, mli, w1u, w1i, b1, w2, b2, wfm, wfh, bf)


def kernel(user_input, item_input, mf_user_emb, mf_item_emb,
           mlp_user_emb, mlp_item_emb, W1, b1, W2, b2, Wf, bf):
    uidx = user_input.astype(jnp.int32).reshape(_NW, _NCHUNK, _CHUNK)
    iidx = item_input.astype(jnp.int32).reshape(_NW, _NCHUNK, _CHUNK)
    mfu, mfi, mlu, mli = _sc_gather(
        uidx, iidx, mf_user_emb, mf_item_emb, mlp_user_emb, mlp_item_emb)
    w1u = W1[:_MLP_D]
    w1i = W1[_MLP_D:]
    wfm = Wf[:_MF_D]
    wfh = Wf[_MF_D:]
    out = _tc_mlp(mfu, mfi, mlu, mli,
                  w1u, w1i, b1.reshape(1, -1), W2, b2.reshape(1, -1),
                  wfm, wfh, bf.reshape(1, 1))
    return out


# speedup vs baseline: 1.4654x; 1.4654x over previous
import functools

import jax
import jax.numpy as jnp
from jax import lax
from jax.experimental import pallas as pl
from jax.experimental.pallas import tpu as pltpu
from jax.experimental.pallas import tpu_sc as plsc

_B = 16384
_MF_D = 32
_MLP_D = 64
_NC, _NS = 2, 16
_NW = _NC * _NS
_BPW = _B // _NW

_TC_BLK = 2048


_BPC = 128
_NCH = _BPW // _BPC


def _sc_gather_kernel(uidx_hbm, iidx_hbm, mfu_hbm, mfi_hbm, mlu_hbm, mli_hbm,
                      out_mfu, out_mfi, out_mlu, out_mli,
                      uidx_v, iidx_v, mfu_v, mfi_v, mlu_v, mli_v, sem):
    wid = lax.axis_index("s") * _NC + lax.axis_index("c")
    base = wid * _BPW
    pltpu.sync_copy(uidx_hbm.at[wid], uidx_v)
    pltpu.sync_copy(iidx_hbm.at[wid], iidx_v)

    for c in range(_NCH):
        cbase = c * _BPC

        def body(g, carry):
            uvec = uidx_v[pl.ds(cbase + g * 16, 16)]
            ivec = iidx_v[pl.ds(cbase + g * 16, 16)]
            for j in range(16):
                u = uvec[j]
                v = ivec[j]
                r = g * 16 + j
                pltpu.async_copy(mfu_hbm.at[pl.ds(u, 1)], mfu_v.at[pl.ds(r, 1)], sem)
                pltpu.async_copy(mfi_hbm.at[pl.ds(v, 1)], mfi_v.at[pl.ds(r, 1)], sem)
                pltpu.async_copy(mlu_hbm.at[pl.ds(u, 1)], mlu_v.at[pl.ds(r, 1)], sem)
                pltpu.async_copy(mli_hbm.at[pl.ds(v, 1)], mli_v.at[pl.ds(r, 1)], sem)
            return carry

        lax.fori_loop(0, _BPC // 16, body, 0)
        pltpu.make_async_copy(out_mfu.at[pl.ds(base, _BPC)], mfu_v, sem).wait()
        pltpu.make_async_copy(out_mfi.at[pl.ds(base, _BPC)], mfi_v, sem).wait()
        pltpu.make_async_copy(out_mlu.at[pl.ds(base, _BPC)], mlu_v, sem).wait()
        pltpu.make_async_copy(out_mli.at[pl.ds(base, _BPC)], mli_v, sem).wait()
        pltpu.sync_copy(mfu_v, out_mfu.at[pl.ds(base + cbase, _BPC)])
        pltpu.sync_copy(mfi_v, out_mfi.at[pl.ds(base + cbase, _BPC)])
        pltpu.sync_copy(mlu_v, out_mlu.at[pl.ds(base + cbase, _BPC)])
        pltpu.sync_copy(mli_v, out_mli.at[pl.ds(base + cbase, _BPC)])


_sc_gather = functools.partial(
    pl.kernel,
    mesh=plsc.VectorSubcoreMesh(core_axis_name="c", subcore_axis_name="s"),
    out_type=[
        jax.ShapeDtypeStruct((_B, _MF_D), jnp.float32),
        jax.ShapeDtypeStruct((_B, _MF_D), jnp.float32),
        jax.ShapeDtypeStruct((_B, _MLP_D), jnp.float32),
        jax.ShapeDtypeStruct((_B, _MLP_D), jnp.float32),
    ],
    scratch_types=[
        pltpu.VMEM((_BPW,), jnp.int32),
        pltpu.VMEM((_BPW,), jnp.int32),
        pltpu.VMEM((_BPC, _MF_D), jnp.float32),
        pltpu.VMEM((_BPC, _MF_D), jnp.float32),
        pltpu.VMEM((_BPC, _MLP_D), jnp.float32),
        pltpu.VMEM((_BPC, _MLP_D), jnp.float32),
        pltpu.SemaphoreType.DMA,
    ],
)(_sc_gather_kernel)


def _tc_mlp_kernel(mfu, mfi, mlu, mli, w1u, w1i, b1, w2, b2, wfm, wfh, bf, out):
    h = jnp.dot(mlu[...], w1u[...], preferred_element_type=jnp.float32)
    h = h + jnp.dot(mli[...], w1i[...], preferred_element_type=jnp.float32)
    h = jnp.maximum(h + b1[...], 0.0)
    h = jnp.dot(h, w2[...], preferred_element_type=jnp.float32) + b2[...]
    h = jnp.maximum(h, 0.0)
    mf = mfu[...] * mfi[...]
    out[...] = (jnp.dot(mf, wfm[...], preferred_element_type=jnp.float32)
                + jnp.dot(h, wfh[...], preferred_element_type=jnp.float32)
                + bf[...])


def _tc_mlp(mfu, mfi, mlu, mli, w1u, w1i, b1, w2, b2, wfm, wfh, bf):
    grid = _B // _TC_BLK
    row_spec = lambda d: pl.BlockSpec((_TC_BLK, d), lambda i: (i, 0))
    full = lambda a: pl.BlockSpec(a.shape, lambda i: (0,) * a.ndim)
    return pl.pallas_call(
        _tc_mlp_kernel,
        grid=(grid,),
        in_specs=[
            row_spec(_MF_D), row_spec(_MF_D), row_spec(_MLP_D), row_spec(_MLP_D),
            full(w1u), full(w1i), full(b1), full(w2), full(b2),
            full(wfm), full(wfh), full(bf),
        ],
        out_specs=pl.BlockSpec((_TC_BLK, 1), lambda i: (i, 0)),
        out_shape=jax.ShapeDtypeStruct((_B, 1), jnp.float32),
    )(mfu, mfi, mlu, mli, w1u, w1i, b1, w2, b2, wfm, wfh, bf)


def kernel(user_input, item_input, mf_user_emb, mf_item_emb,
           mlp_user_emb, mlp_item_emb, W1, b1, W2, b2, Wf, bf):
    uidx = user_input.astype(jnp.int32).reshape(_NW, _BPW)
    iidx = item_input.astype(jnp.int32).reshape(_NW, _BPW)
    mfu, mfi, mlu, mli = _sc_gather(
        uidx, iidx, mf_user_emb, mf_item_emb, mlp_user_emb, mlp_item_emb)
    w1u = W1[:_MLP_D]
    w1i = W1[_MLP_D:]
    wfm = Wf[:_MF_D]
    wfh = Wf[_MF_D:]
    out = _tc_mlp(mfu, mfi, mlu, mli,
                  w1u, w1i, b1.reshape(1, -1), W2, b2.reshape(1, -1),
                  wfm, wfh, bf.reshape(1, 1))
    return out
